# Initial kernel scaffold; baseline (speedup 1.0000x reference)
import jax
import jax.numpy as jnp
from jax.experimental import pallas as pl


def kernel(x, W):
    raise NotImplementedError("write your pallas kernel here")


# speedup vs baseline: 1.4896x; 1.4896x over previous
import functools

import jax
import jax.numpy as jnp
from jax import lax
from jax.experimental import pallas as pl
from jax.experimental.pallas import tpu as pltpu
from jax.experimental.pallas import tpu_sc as plsc

_NC = 2
_NS = 16
_NW = _NC * _NS

_CHUNK = 2560


@functools.partial(jax.jit, static_argnames=("n", "d"))
def _sc_gather(idx, table, n, d):
    b_per_w = n // _NW
    steps = b_per_w // _CHUNK
    mesh = plsc.VectorSubcoreMesh(core_axis_name="c", subcore_axis_name="s")

    @functools.partial(
        pl.kernel,
        mesh=mesh,
        out_type=jax.ShapeDtypeStruct((n, d), jnp.float32),
        scratch_types=[
            pltpu.VMEM((_CHUNK,), jnp.int32),
            pltpu.VMEM((_CHUNK, d), jnp.float32),
            pltpu.SemaphoreType.DMA,
        ],
        compiler_params=pltpu.CompilerParams(use_tc_tiling_on_sc=False),
    )
    def k(idx_hbm, table_hbm, out_hbm, idx_v, rows_v, sem):
        wid = lax.axis_index("s") * _NC + lax.axis_index("c")
        base = wid * b_per_w

        def body(i, carry):
            off = base + i * _CHUNK
            pltpu.sync_copy(idx_hbm.at[pl.ds(off, _CHUNK)], idx_v)
            pltpu.async_copy(table_hbm.at[idx_v], rows_v, sem).wait()
            pltpu.sync_copy(rows_v, out_hbm.at[pl.ds(off, _CHUNK), :])
            return carry

        lax.fori_loop(0, steps, body, 0, unroll=False)

    return k(idx, table)


def kernel(x, W):
    b, h = x.shape
    v, d = W.shape
    n = b * h
    idx = x.reshape(n).astype(jnp.int32)
    out = _sc_gather(idx, W, n, d)
    return out.reshape(b, h, d)


# speedup vs baseline: 1.5043x; 1.0098x over previous
import functools

import jax
import jax.numpy as jnp
from jax import lax
from jax.experimental import pallas as pl
from jax.experimental.pallas import tpu as pltpu
from jax.experimental.pallas import tpu_sc as plsc

_NC = 2
_NS = 16
_NW = _NC * _NS

_CHUNK = 1024
_NBUF = 3


@functools.partial(jax.jit, static_argnames=("n", "d"))
def _sc_gather(idx, table, n, d):
    b_per_w = n // _NW
    steps = b_per_w // _CHUNK
    mesh = plsc.VectorSubcoreMesh(core_axis_name="c", subcore_axis_name="s")

    @functools.partial(
        pl.kernel,
        mesh=mesh,
        out_type=jax.ShapeDtypeStruct((n, d), jnp.float32),
        scratch_types=[
            pltpu.VMEM((b_per_w,), jnp.int32),
            [pltpu.VMEM((_CHUNK, d), jnp.float32) for _ in range(_NBUF)],
            [pltpu.SemaphoreType.DMA for _ in range(_NBUF)],
            [pltpu.SemaphoreType.DMA for _ in range(_NBUF)],
        ],
        compiler_params=pltpu.CompilerParams(use_tc_tiling_on_sc=False),
    )
    def k(idx_hbm, table_hbm, out_hbm, idx_v, rows, gsems, ssems):
        wid = lax.axis_index("s") * _NC + lax.axis_index("c")
        base = wid * b_per_w
        pltpu.sync_copy(idx_hbm.at[pl.ds(base, b_per_w)], idx_v)

        def start_gather(i, b):
            return pltpu.async_copy(
                table_hbm.at[idx_v.at[pl.ds(i * _CHUNK, _CHUNK)]],
                rows[b], gsems[b])

        gathers = [None] * steps
        stores = [None] * steps
        for b in range(min(_NBUF, steps)):
            gathers[b] = start_gather(b, b)
        for i in range(steps):
            b = i % _NBUF
            gathers[i].wait()
            stores[i] = pltpu.async_copy(
                rows[b], out_hbm.at[pl.ds(base + i * _CHUNK, _CHUNK), :],
                ssems[b])
            nxt = i + _NBUF
            if nxt < steps:
                stores[i].wait()
                gathers[nxt] = start_gather(nxt, b)
        for i in range(max(0, steps - _NBUF), steps):
            stores[i].wait()

    return k(idx, table)


def kernel(x, W):
    b, h = x.shape
    v, d = W.shape
    n = b * h
    idx = x.reshape(n).astype(jnp.int32)
    out = _sc_gather(idx, W, n, d)
    return out.reshape(b, h, d)
